# Initial kernel scaffold; baseline (speedup 1.0000x reference)
import jax
import jax.numpy as jnp
from jax.experimental import pallas as pl


def kernel(x, Wr, W1, W2):
    raise NotImplementedError("write your pallas kernel here")


# speedup vs baseline: 1.0376x; 1.0376x over previous
import functools

import jax
import jax.numpy as jnp
from jax.experimental import pallas as pl
from jax.experimental.pallas import tpu as pltpu

S, D, E, TOPK = 2048, 768, 8, 2
H = 4 * D
TB = 128
HC = 768
NT = S // TB
NH = H // HC


def _dense_body(x_ref, wr_ref, w1_ref, w2_ref, o_ref, wblk, acc):
    e = pl.program_id(1)
    hc = pl.program_id(2)
    lane = jax.lax.broadcasted_iota(jnp.int32, (TB, E), 1)

    @pl.when((e == 0) & (hc == 0))
    def _():
        xb = x_ref[...]
        logits = jax.lax.dot_general(xb, wr_ref[...], (((1,), (1,)), ((), ())))
        m0 = jnp.max(logits, axis=1, keepdims=True)
        i0 = jnp.min(jnp.where(logits == m0, lane, E), axis=1, keepdims=True)
        l2 = jnp.where(lane == i0, -jnp.inf, logits)
        m1 = jnp.max(l2, axis=1, keepdims=True)
        i1 = jnp.min(jnp.where(l2 == m1, lane, E), axis=1, keepdims=True)
        w0 = jax.nn.sigmoid(m0 - m1)
        wblk[...] = (jnp.where(lane == i0, w0, 0.0)
                     + jnp.where(lane == i1, 1.0 - w0, 0.0))
        acc[...] = jnp.zeros_like(acc)

    xb = x_ref[...]
    h = jax.lax.dot_general(xb, w1_ref[0], (((1,), (1,)), ((), ())))
    h = 0.5 * h * (1.0 + jax.lax.erf(h * 0.7071067811865476))
    y = jax.lax.dot_general(h, w2_ref[0], (((1,), (1,)), ((), ())))
    we = jnp.sum(jnp.where(lane == e, wblk[...], 0.0), axis=1, keepdims=True)
    acc[...] += we * y

    @pl.when((e == E - 1) & (hc == NH - 1))
    def _():
        o_ref[...] = acc[...]


@jax.jit
def _moe(xf, Wr, W1, W2):
    return pl.pallas_call(
        _dense_body,
        grid=(NT, E, NH),
        in_specs=[
            pl.BlockSpec((TB, D), lambda t, e, hc: (t, 0)),
            pl.BlockSpec((E, D), lambda t, e, hc: (0, 0)),
            pl.BlockSpec((1, HC, D), lambda t, e, hc: (e, hc, 0)),
            pl.BlockSpec((1, D, HC), lambda t, e, hc: (e, 0, hc)),
        ],
        out_specs=pl.BlockSpec((TB, D), lambda t, e, hc: (t, 0)),
        out_shape=jax.ShapeDtypeStruct((S, D), jnp.float32),
        scratch_shapes=[
            pltpu.VMEM((TB, E), jnp.float32),
            pltpu.VMEM((TB, D), jnp.float32),
        ],
        compiler_params=pltpu.CompilerParams(
            dimension_semantics=("parallel", "arbitrary", "arbitrary"),
        ),
    )(xf, Wr, W1, W2)


def kernel(x, Wr, W1, W2):
    b, s, d = x.shape
    out = _moe(x.reshape(s, d), Wr, W1, W2)
    return out.reshape(b, s, d)


# speedup vs baseline: 2.7166x; 2.6183x over previous
import functools

import jax
import jax.numpy as jnp
from jax import lax
from jax.experimental import pallas as pl
from jax.experimental.pallas import tpu as pltpu
from jax.experimental.pallas import tpu_sc as plsc

S, D, E = 2048, 768, 8
H = 4 * D
P = 2 * S
M = 128
G = P // M + E
GM = G * M
HC = 1536
NH = H // HC
NC, NS = 2, 16
NW = NC * NS
TPW = S // NW
SPW = GM // NW

_mesh = plsc.VectorSubcoreMesh(
    core_axis_name="c", subcore_axis_name="s", num_cores=NC, num_subcores=NS)


def _wid():
    return lax.axis_index("s") * NC + lax.axis_index("c")


def _router_body(x_ref, wr_ref, eids_ref, wts_ref):
    logits = lax.dot_general(x_ref[...], wr_ref[...],
                             (((1,), (1,)), ((), ())))
    lane = lax.broadcasted_iota(jnp.int32, (S, E), 1)
    m0 = jnp.max(logits, axis=1, keepdims=True)
    i0 = jnp.min(jnp.where(logits == m0, lane, E), axis=1, keepdims=True)
    l2 = jnp.where(lane == i0, -jnp.inf, logits)
    m1 = jnp.max(l2, axis=1, keepdims=True)
    i1 = jnp.min(jnp.where(l2 == m1, lane, E), axis=1, keepdims=True)
    w0 = jax.nn.sigmoid(m0 - m1)
    eids_ref[...] = (jnp.where(lane == 0, i0, 0)
                     + jnp.where(lane == 1, i1, 0)).astype(jnp.int32)
    wts_ref[...] = (jnp.where(lane == 0, w0, 0.0)
                    + jnp.where(lane == 1, 1.0 - w0, 0.0))


_router = pl.pallas_call(
    _router_body,
    out_shape=(jax.ShapeDtypeStruct((S, E), jnp.int32),
               jax.ShapeDtypeStruct((S, E), jnp.float32)),
)


@functools.partial(
    pl.kernel, mesh=_mesh,
    compiler_params=pltpu.CompilerParams(needs_layout_passes=False),
    out_type=jax.ShapeDtypeStruct((NW, 16), jnp.int32),
    scratch_types=[pltpu.VMEM((TPW, E), jnp.int32),
                   pltpu.VMEM((16,), jnp.int32)])
def _b1a(eids_hbm, counts_hbm, ev_v, cnt_v):
    wid = _wid()
    tbase = wid * TPW
    pltpu.sync_copy(eids_hbm.at[pl.ds(tbase, TPW)], ev_v)
    it = lax.iota(jnp.int32, 16)
    acc = [jnp.zeros((16,), jnp.int32) for _ in range(E)]
    for k in range(2):
        cols = jnp.full((16,), k, jnp.int32)
        for j in range(TPW // 16):
            ev = plsc.load_gather(ev_v, [j * 16 + it, cols])
            for e in range(E):
                acc[e] = acc[e] + jnp.where(ev == e, 1, 0)
    cntvec = jnp.zeros((16,), jnp.int32)
    for e in range(E):
        cntvec = jnp.where(it == e, jnp.sum(acc[e]), cntvec)
    cnt_v[...] = cntvec
    pltpu.sync_copy(cnt_v, counts_hbm.at[wid])


@functools.partial(
    pl.kernel, mesh=_mesh,
    compiler_params=pltpu.CompilerParams(needs_layout_passes=False),
    out_type=(jax.ShapeDtypeStruct((GM,), jnp.int32),
              jax.ShapeDtypeStruct((P,), jnp.int32),
              jax.ShapeDtypeStruct((64,), jnp.int32),
              jax.ShapeDtypeStruct((64,), jnp.int32)),
    scratch_types=[pltpu.VMEM((NW, 16), jnp.int32),
                   pltpu.VMEM((TPW, E), jnp.int32),
                   pltpu.SMEM((E,), jnp.int32),
                   pltpu.VMEM((2, TPW), jnp.int32),
                   pltpu.VMEM((2 * TPW,), jnp.int32),
                   pltpu.VMEM((2 * TPW,), jnp.int32),
                   pltpu.VMEM((64,), jnp.int32),
                   pltpu.VMEM((64,), jnp.int32),
                   pltpu.SemaphoreType.DMA])
def _b1b(eids_hbm, counts_hbm, gidx_hbm, inv_hbm, be_hbm, ba_hbm,
         cnts_v, ev_v, offs_s, inv_v, idx1d, tok1d, be_v, ba_v, sem):
    wid = _wid()
    tbase = wid * TPW
    pltpu.sync_copy(counts_hbm, cnts_v)
    pltpu.sync_copy(eids_hbm.at[pl.ds(tbase, TPW)], ev_v)

    tot_vec = jnp.zeros((16,), jnp.int32)
    mine_vec = jnp.zeros((16,), jnp.int32)
    for t in range(NW):
        row = cnts_v[t]
        tot_vec = tot_vec + row
        mine_vec = mine_vec + jnp.where(t < wid, row, 0)
    cum = jnp.int32(0)
    nbs = []
    for e in range(E):
        tot = tot_vec[e]
        mine = mine_vec[e]
        offs_s[e] = cum * M + mine
        nb = (tot + M - 1) // M
        nbs.append(nb)
        cum = cum + nb

    it = lax.iota(jnp.int32, 16)
    for k in range(2):
        cols = jnp.full((16,), k, jnp.int32)
        for j in range(TPW // 16):
            ev = plsc.load_gather(ev_v, [j * 16 + it, cols])
            slot = jnp.zeros((16,), jnp.int32)
            for e in range(E):
                m = ev == e
                mi = jnp.where(m, 1, 0)
                pc = plsc.cumsum(mi)
                off_e = offs_s[e]
                slot = jnp.where(m, off_e + pc - 1, slot)
                offs_s[e] = off_e + jnp.sum(mi)
            pos = k * TPW + j * 16
            idx1d[pl.ds(pos, 16)] = slot
            tok1d[pl.ds(pos, 16)] = tbase + j * 16 + it
            inv_v[k, pl.ds(j * 16, 16)] = slot
    pltpu.async_copy(tok1d, gidx_hbm.at[idx1d], sem).wait()
    pltpu.sync_copy(inv_v.at[0], inv_hbm.at[pl.ds(tbase, TPW)])
    pltpu.sync_copy(inv_v.at[1], inv_hbm.at[pl.ds(S + tbase, TPW)])

    @pl.when(wid == 0)
    def _():
        cnb = [jnp.int32(0)]
        for e in range(E):
            cnb.append(cnb[-1] + nbs[e])
        for v in range(4):
            g_ids = v * 16 + it
            bevec = jnp.zeros((16,), jnp.int32)
            for e in range(1, E):
                bevec = jnp.where(g_ids >= cnb[e], jnp.int32(e), bevec)
            be_v[pl.ds(v * 16, 16)] = bevec
            ba_v[pl.ds(v * 16, 16)] = jnp.where(g_ids < cnb[E], 1, 0
                                                ).astype(jnp.int32)
        pltpu.sync_copy(be_v, be_hbm)
        pltpu.sync_copy(ba_v, ba_hbm)


@functools.partial(
    pl.kernel, mesh=_mesh,
    compiler_params=pltpu.CompilerParams(needs_layout_passes=False),
    out_type=jax.ShapeDtypeStruct((GM, D), jnp.float32),
    scratch_types=[pltpu.VMEM((SPW // 2,), jnp.int32),
                   pltpu.VMEM((SPW // 2,), jnp.int32),
                   pltpu.VMEM((SPW // 2, D), jnp.float32),
                   pltpu.SemaphoreType.DMA])
def _b2(gidx_hbm, x_hbm, xs_hbm, gidr, gidc, rows_v, sem):
    wid = _wid()
    half = SPW // 2
    for c in range(2):
        base = wid * SPW + c * half
        pltpu.sync_copy(gidx_hbm.at[pl.ds(base, half)], gidr)
        for v in range(half // 16):
            g = gidr[pl.ds(v * 16, 16)]
            gidc[pl.ds(v * 16, 16)] = jnp.clip(g, 0, S - 1)
        pltpu.async_copy(x_hbm.at[gidc], rows_v, sem).wait()
        pltpu.sync_copy(rows_v, xs_hbm.at[pl.ds(base, half)])


def _mlp_body(be_ref, ba_ref, xs_ref, w1_ref, w2_ref, ys_ref, acc_ref):
    g = pl.program_id(0)
    hc = pl.program_id(1)

    @pl.when(ba_ref[g] > 0)
    def _():
        xb = xs_ref[...]
        h = lax.dot_general(xb, w1_ref[0], (((1,), (1,)), ((), ())))
        h = 0.5 * h * (1.0 + lax.erf(h * 0.7071067811865476))
        y = lax.dot_general(h, w2_ref[0], (((1,), (1,)), ((), ())))

        @pl.when(hc == 0)
        def _():
            acc_ref[...] = y

        @pl.when(hc > 0)
        def _():
            acc_ref[...] += y

        @pl.when(hc == NH - 1)
        def _():
            ys_ref[...] = acc_ref[...]


_mlp = pl.pallas_call(
    _mlp_body,
    grid_spec=pltpu.PrefetchScalarGridSpec(
        num_scalar_prefetch=2,
        grid=(G, NH),
        in_specs=[
            pl.BlockSpec((M, D), lambda g, hc, be, ba: (g, 0)),
            pl.BlockSpec((1, HC, D), lambda g, hc, be, ba: (be[g], hc, 0)),
            pl.BlockSpec((1, D, HC), lambda g, hc, be, ba: (be[g], 0, hc)),
        ],
        out_specs=pl.BlockSpec((M, D), lambda g, hc, be, ba: (g, 0)),
        scratch_shapes=[pltpu.VMEM((M, D), jnp.float32)],
    ),
    out_shape=jax.ShapeDtypeStruct((GM, D), jnp.float32),
    compiler_params=pltpu.CompilerParams(
        dimension_semantics=("arbitrary", "arbitrary"),
    ),
)


@functools.partial(
    pl.kernel, mesh=_mesh,
    compiler_params=pltpu.CompilerParams(needs_layout_passes=False),
    out_type=jax.ShapeDtypeStruct((S, D), jnp.float32),
    scratch_types=[pltpu.VMEM((TPW // 2,), jnp.int32),
                   pltpu.VMEM((TPW // 2,), jnp.int32),
                   pltpu.VMEM((TPW // 2, E), jnp.float32),
                   pltpu.VMEM((TPW // 2, D), jnp.float32),
                   pltpu.VMEM((TPW // 2, D), jnp.float32),
                   pltpu.VMEM((TPW // 2, D), jnp.float32),
                   pltpu.SemaphoreType.DMA])
def _combine(inv_hbm, wts_hbm, ys_hbm, out_hbm,
             s0, s1, wrow_v, y0, y1, o_v, sem):
    wid = _wid()
    half = TPW // 2
    for c in range(2):
        t0 = wid * TPW + c * half
        pltpu.sync_copy(inv_hbm.at[pl.ds(t0, half)], s0)
        pltpu.sync_copy(inv_hbm.at[pl.ds(S + t0, half)], s1)
        pltpu.sync_copy(wts_hbm.at[pl.ds(t0, half)], wrow_v)
        pltpu.async_copy(ys_hbm.at[s0], y0, sem).wait()
        pltpu.async_copy(ys_hbm.at[s1], y1, sem).wait()

        zeros16 = jnp.zeros((16,), jnp.int32)

        def body(i, _):
            a = plsc.load_gather(wrow_v, [jnp.full((16,), i, jnp.int32),
                                          zeros16])
            b = plsc.load_gather(wrow_v, [jnp.full((16,), i, jnp.int32),
                                          zeros16 + 1])
            for jj in range(D // 16):
                sl = pl.ds(jj * 16, 16)
                o_v[i, sl] = a * y0[i, sl] + b * y1[i, sl]
            return 0

        lax.fori_loop(0, half, body, 0)
        pltpu.sync_copy(o_v, out_hbm.at[pl.ds(t0, half)])


@jax.jit
def _moe(xf, Wr, W1, W2):
    eids, wts = _router(xf, Wr)
    counts = _b1a(eids)
    gidx, inv, be, ba = _b1b(eids, counts)
    xs = _b2(gidx, xf)
    ys = _mlp(be, ba, xs, W1, W2)
    return _combine(inv, wts, ys)


def kernel(x, Wr, W1, W2):
    b, s, d = x.shape
    out = _moe(x.reshape(s, d), Wr, W1, W2)
    return out.reshape(b, s, d)


# speedup vs baseline: 3.3316x; 1.2264x over previous
import functools

import jax
import jax.numpy as jnp
from jax import lax
from jax.experimental import pallas as pl
from jax.experimental.pallas import tpu as pltpu
from jax.experimental.pallas import tpu_sc as plsc

S, D, E = 2048, 768, 8
H = 4 * D
P = 2 * S
M = 128
G = P // M + E
GM = G * M
HC = 1536
NH = H // HC
NC, NS = 2, 16
NW = NC * NS
TPW = S // NW
NG = S // 16

_mesh = plsc.VectorSubcoreMesh(
    core_axis_name="c", subcore_axis_name="s", num_cores=NC, num_subcores=NS)

_sc_params = pltpu.CompilerParams(needs_layout_passes=False)


def _wid():
    return lax.axis_index("s") * NC + lax.axis_index("c")


def _router_body(x_ref, wr_ref, eids_ref, wts_ref):
    logits = lax.dot_general(x_ref[...], wr_ref[...],
                             (((1,), (1,)), ((), ())))
    lane = lax.broadcasted_iota(jnp.int32, (S, E), 1)
    m0 = jnp.max(logits, axis=1, keepdims=True)
    i0 = jnp.min(jnp.where(logits == m0, lane, E), axis=1, keepdims=True)
    l2 = jnp.where(lane == i0, -jnp.inf, logits)
    m1 = jnp.max(l2, axis=1, keepdims=True)
    i1 = jnp.min(jnp.where(l2 == m1, lane, E), axis=1, keepdims=True)
    w0 = jax.nn.sigmoid(m0 - m1)
    eids_ref[...] = (jnp.where(lane == 0, i0, 0)
                     + jnp.where(lane == 1, i1, 0)).astype(jnp.int32)
    wts_ref[...] = (jnp.where(lane == 0, w0, 0.0)
                    + jnp.where(lane == 1, 1.0 - w0, 0.0))


_router = pl.pallas_call(
    _router_body,
    out_shape=(jax.ShapeDtypeStruct((S, E), jnp.int32),
               jax.ShapeDtypeStruct((S, E), jnp.float32)),
)


@functools.partial(
    pl.kernel, mesh=_mesh,
    compiler_params=_sc_params,
    out_type=(jax.ShapeDtypeStruct((GM, D), jnp.float32),
              jax.ShapeDtypeStruct((P,), jnp.int32),
              jax.ShapeDtypeStruct((64,), jnp.int32),
              jax.ShapeDtypeStruct((64,), jnp.int32)),
    scratch_types=[pltpu.VMEM((P,), jnp.int32),
                   pltpu.VMEM((TPW, D), jnp.float32),
                   pltpu.SMEM((E,), jnp.int32),
                   pltpu.VMEM((TPW,), jnp.int32),
                   pltpu.VMEM((TPW,), jnp.int32),
                   pltpu.VMEM((64,), jnp.int32),
                   pltpu.VMEM((64,), jnp.int32),
                   pltpu.SemaphoreType.DMA,
                   pltpu.SemaphoreType.DMA])
def _dispatch(ep_hbm, x_hbm, xs_hbm, inv_hbm, be_hbm, ba_hbm,
              ev_v, xrow_v, offs_s, s0_v, s1_v, be_v, ba_v, sem, sem2):
    wid = _wid()
    tbase = wid * TPW
    cx = pltpu.async_copy(x_hbm.at[pl.ds(tbase, TPW)], xrow_v, sem2)
    pltpu.sync_copy(ep_hbm, ev_v)
    it = lax.iota(jnp.int32, 16)
    gstart = wid * (TPW // 16)

    def hist_body(g, carry):
        accs = list(carry)
        rows = g * 16 + it
        before = (g < gstart).astype(jnp.int32)
        for k in range(2):
            ev = plsc.load_gather(ev_v, [rows * 2 + k])
            for e in range(E):
                cnt = jnp.where(ev == e, 1, 0)
                accs[e] = accs[e] + cnt
                accs[E + e] = accs[E + e] + before * cnt
        return tuple(accs)

    zero = jnp.zeros((16,), jnp.int32)
    accs = lax.fori_loop(0, NG, hist_body, (zero,) * (2 * E))

    cum = jnp.int32(0)
    nbs = []
    for e in range(E):
        tot = jnp.sum(accs[e])
        mine = jnp.sum(accs[E + e])
        offs_s[e] = cum * M + mine
        nb = (tot + M - 1) // M
        nbs.append(nb)
        cum = cum + nb

    for k in range(2):
        dst = s0_v if k == 0 else s1_v
        for j in range(TPW // 16):
            ev = plsc.load_gather(ev_v, [(tbase + j * 16 + it) * 2 + k])
            slot = jnp.zeros((16,), jnp.int32)
            for e in range(E):
                m = ev == e
                mi = jnp.where(m, 1, 0)
                pc = plsc.cumsum(mi)
                off_e = offs_s[e]
                slot = jnp.where(m, off_e + pc - 1, slot)
                offs_s[e] = off_e + jnp.sum(mi)
            dst[pl.ds(j * 16, 16)] = slot

    cx.wait()
    c0 = pltpu.async_copy(xrow_v, xs_hbm.at[s0_v], sem)
    c1 = pltpu.async_copy(xrow_v, xs_hbm.at[s1_v], sem2)
    pltpu.sync_copy(s0_v, inv_hbm.at[pl.ds(tbase, TPW)])
    pltpu.sync_copy(s1_v, inv_hbm.at[pl.ds(S + tbase, TPW)])
    c0.wait()
    c1.wait()

    @pl.when(wid == 0)
    def _():
        cnb = [jnp.int32(0)]
        for e in range(E):
            cnb.append(cnb[-1] + nbs[e])
        for v in range(4):
            g_ids = v * 16 + it
            bevec = jnp.zeros((16,), jnp.int32)
            for e in range(1, E):
                bevec = jnp.where(g_ids >= cnb[e], jnp.int32(e), bevec)
            be_v[pl.ds(v * 16, 16)] = bevec
            ba_v[pl.ds(v * 16, 16)] = jnp.where(g_ids < cnb[E], 1, 0
                                                ).astype(jnp.int32)
        pltpu.sync_copy(be_v, be_hbm)
        pltpu.sync_copy(ba_v, ba_hbm)


def _mlp_body(be_ref, ba_ref, xs_ref, w1_ref, w2_ref, ys_ref, acc_ref):
    g = pl.program_id(0)
    hc = pl.program_id(1)

    @pl.when(ba_ref[g] > 0)
    def _():
        xb = xs_ref[...]
        h = lax.dot_general(xb, w1_ref[0], (((1,), (1,)), ((), ())))
        h = 0.5 * h * (1.0 + lax.erf(h * 0.7071067811865476))
        y = lax.dot_general(h, w2_ref[0], (((1,), (1,)), ((), ())))

        @pl.when(hc == 0)
        def _():
            acc_ref[...] = y

        @pl.when(hc > 0)
        def _():
            acc_ref[...] += y

        @pl.when(hc == NH - 1)
        def _():
            ys_ref[...] = acc_ref[...]


_mlp = pl.pallas_call(
    _mlp_body,
    grid_spec=pltpu.PrefetchScalarGridSpec(
        num_scalar_prefetch=2,
        grid=(G, NH),
        in_specs=[
            pl.BlockSpec((M, D), lambda g, hc, be, ba: (g, 0)),
            pl.BlockSpec((1, HC, D), lambda g, hc, be, ba: (be[g], hc, 0)),
            pl.BlockSpec((1, D, HC), lambda g, hc, be, ba: (be[g], 0, hc)),
        ],
        out_specs=pl.BlockSpec((M, D), lambda g, hc, be, ba: (g, 0)),
        scratch_shapes=[pltpu.VMEM((M, D), jnp.float32)],
    ),
    out_shape=jax.ShapeDtypeStruct((GM, D), jnp.float32),
    compiler_params=pltpu.CompilerParams(
        dimension_semantics=("arbitrary", "arbitrary"),
    ),
)


@functools.partial(
    pl.kernel, mesh=_mesh,
    compiler_params=_sc_params,
    out_type=jax.ShapeDtypeStruct((S, D), jnp.float32),
    scratch_types=[pltpu.VMEM((TPW // 2,), jnp.int32),
                   pltpu.VMEM((TPW // 2,), jnp.int32),
                   pltpu.VMEM((TPW // 2,), jnp.float32),
                   pltpu.VMEM((TPW // 2,), jnp.float32),
                   pltpu.VMEM((TPW // 2, D), jnp.float32),
                   pltpu.VMEM((TPW // 2, D), jnp.float32),
                   pltpu.VMEM((TPW // 2, D), jnp.float32),
                   pltpu.SemaphoreType.DMA])
def _combine(inv_hbm, w0_hbm, w1_hbm, ys_hbm, out_hbm,
             s0, s1, w0_v, w1_v, y0, y1, o_v, sem):
    wid = _wid()
    half = TPW // 2
    for c in range(2):
        t0 = wid * TPW + c * half
        pltpu.sync_copy(inv_hbm.at[pl.ds(t0, half)], s0)
        pltpu.sync_copy(inv_hbm.at[pl.ds(S + t0, half)], s1)
        c0 = pltpu.async_copy(ys_hbm.at[s0], y0, sem)
        c1 = pltpu.async_copy(ys_hbm.at[s1], y1, sem)
        pltpu.sync_copy(w0_hbm.at[pl.ds(t0, half)], w0_v)
        pltpu.sync_copy(w1_hbm.at[pl.ds(t0, half)], w1_v)
        c0.wait()
        c1.wait()

        def body(i, _):
            a = plsc.load_gather(w0_v, [jnp.full((16,), i, jnp.int32)])
            b = plsc.load_gather(w1_v, [jnp.full((16,), i, jnp.int32)])
            for jj in range(D // 16):
                sl = pl.ds(jj * 16, 16)
                o_v[i, sl] = a * y0[i, sl] + b * y1[i, sl]
            return 0

        lax.fori_loop(0, half, body, 0)
        pltpu.sync_copy(o_v, out_hbm.at[pl.ds(t0, half)])


@jax.jit
def _moe(xf, Wr, W1, W2):
    eids, wts = _router(xf, Wr)
    ep = eids[:, :2].reshape(P)
    xs, inv, be, ba = _dispatch(ep, xf)
    ys = _mlp(be, ba, xs, W1, W2)
    return _combine(inv, wts[:, 0], wts[:, 1], ys)


def kernel(x, Wr, W1, W2):
    b, s, d = x.shape
    out = _moe(x.reshape(s, d), Wr, W1, W2)
    return out.reshape(b, s, d)


# speedup vs baseline: 4.6321x; 1.3903x over previous
import functools

import jax
import jax.numpy as jnp
from jax import lax
from jax.experimental import pallas as pl
from jax.experimental.pallas import tpu as pltpu
from jax.experimental.pallas import tpu_sc as plsc

S, D, E = 2048, 768, 8
H = 4 * D
P = 2 * S
M = 128
G = P // M + E
GM = G * M
HC = H
NH = H // HC
NC, NS = 2, 16
NW = NC * NS
TPW = S // NW
NG = S // 16

_mesh = plsc.VectorSubcoreMesh(
    core_axis_name="c", subcore_axis_name="s", num_cores=NC, num_subcores=NS)

_sc_params = pltpu.CompilerParams(needs_layout_passes=False)


def _wid():
    return lax.axis_index("s") * NC + lax.axis_index("c")


def _router_body(x_ref, wr_ref, eids_ref, wts_ref):
    logits = lax.dot_general(x_ref[...], wr_ref[...],
                             (((1,), (1,)), ((), ())))
    lane = lax.broadcasted_iota(jnp.int32, (S, E), 1)
    m0 = jnp.max(logits, axis=1, keepdims=True)
    i0 = jnp.min(jnp.where(logits == m0, lane, E), axis=1, keepdims=True)
    l2 = jnp.where(lane == i0, -jnp.inf, logits)
    m1 = jnp.max(l2, axis=1, keepdims=True)
    i1 = jnp.min(jnp.where(l2 == m1, lane, E), axis=1, keepdims=True)
    w0 = jax.nn.sigmoid(m0 - m1)
    eids_ref[...] = (jnp.where(lane == 0, i0, 0)
                     + jnp.where(lane == 1, i1, 0)).astype(jnp.int32)
    wts_ref[...] = (jnp.where(lane == 0, w0, 0.0)
                    + jnp.where(lane == 1, 1.0 - w0, 0.0))


_router = pl.pallas_call(
    _router_body,
    out_shape=(jax.ShapeDtypeStruct((S, E), jnp.int32),
               jax.ShapeDtypeStruct((S, E), jnp.float32)),
)


@functools.partial(
    pl.kernel, mesh=_mesh,
    compiler_params=_sc_params,
    out_type=(jax.ShapeDtypeStruct((GM, D), jnp.float32),
              jax.ShapeDtypeStruct((P,), jnp.int32),
              jax.ShapeDtypeStruct((64,), jnp.int32),
              jax.ShapeDtypeStruct((64,), jnp.int32)),
    scratch_types=[pltpu.VMEM((P,), jnp.int32),
                   pltpu.VMEM((TPW, D), jnp.float32),
                   pltpu.SMEM((E,), jnp.int32),
                   pltpu.VMEM((TPW,), jnp.int32),
                   pltpu.VMEM((TPW,), jnp.int32),
                   pltpu.VMEM((64,), jnp.int32),
                   pltpu.VMEM((64,), jnp.int32),
                   pltpu.SemaphoreType.DMA,
                   pltpu.SemaphoreType.DMA])
def _dispatch(ep_hbm, x_hbm, xs_hbm, inv_hbm, be_hbm, ba_hbm,
              ev_v, xrow_v, offs_s, s0_v, s1_v, be_v, ba_v, sem, sem2):
    wid = _wid()
    tbase = wid * TPW
    cx = pltpu.async_copy(x_hbm.at[pl.ds(tbase, TPW)], xrow_v, sem2)
    pltpu.sync_copy(ep_hbm, ev_v)
    it = lax.iota(jnp.int32, 16)
    gstart = wid * (TPW // 16)

    def hist_body(g, carry):
        accs = list(carry)
        rows = g * 16 + it
        before = (g < gstart).astype(jnp.int32)
        for k in range(2):
            ev = plsc.load_gather(ev_v, [rows * 2 + k])
            for e in range(E):
                cnt = jnp.where(ev == e, 1, 0)
                accs[e] = accs[e] + cnt
                accs[E + e] = accs[E + e] + before * cnt
        return tuple(accs)

    zero = jnp.zeros((16,), jnp.int32)
    accs = lax.fori_loop(0, NG, hist_body, (zero,) * (2 * E))

    cum = jnp.int32(0)
    nbs = []
    for e in range(E):
        tot = jnp.sum(accs[e])
        mine = jnp.sum(accs[E + e])
        offs_s[e] = cum * M + mine
        nb = (tot + M - 1) // M
        nbs.append(nb)
        cum = cum + nb

    for k in range(2):
        dst = s0_v if k == 0 else s1_v
        for j in range(TPW // 16):
            ev = plsc.load_gather(ev_v, [(tbase + j * 16 + it) * 2 + k])
            slot = jnp.zeros((16,), jnp.int32)
            for e in range(E):
                m = ev == e
                mi = jnp.where(m, 1, 0)
                pc = plsc.cumsum(mi)
                off_e = offs_s[e]
                slot = jnp.where(m, off_e + pc - 1, slot)
                offs_s[e] = off_e + jnp.sum(mi)
            dst[pl.ds(j * 16, 16)] = slot

    cx.wait()
    c0 = pltpu.async_copy(xrow_v, xs_hbm.at[s0_v], sem)
    c1 = pltpu.async_copy(xrow_v, xs_hbm.at[s1_v], sem2)
    pltpu.sync_copy(s0_v, inv_hbm.at[pl.ds(tbase, TPW)])
    pltpu.sync_copy(s1_v, inv_hbm.at[pl.ds(S + tbase, TPW)])
    c0.wait()
    c1.wait()

    @pl.when(wid == 0)
    def _():
        cnb = [jnp.int32(0)]
        for e in range(E):
            cnb.append(cnb[-1] + nbs[e])
        for v in range(4):
            g_ids = v * 16 + it
            bevec = jnp.zeros((16,), jnp.int32)
            for e in range(1, E):
                bevec = jnp.where(g_ids >= cnb[e], jnp.int32(e), bevec)
            be_v[pl.ds(v * 16, 16)] = bevec
            ba_v[pl.ds(v * 16, 16)] = jnp.where(g_ids < cnb[E], 1, 0
                                                ).astype(jnp.int32)
        pltpu.sync_copy(be_v, be_hbm)
        pltpu.sync_copy(ba_v, ba_hbm)


def _mlp_body(be_ref, ba_ref, xs_ref, w1_ref, w2_ref, ys_ref):
    g = pl.program_id(0)

    @pl.when(ba_ref[g] > 0)
    def _():
        xb = xs_ref[...]
        h = lax.dot_general(xb, w1_ref[0], (((1,), (1,)), ((), ())))
        h = 0.5 * h * (1.0 + lax.erf(h * 0.7071067811865476))
        ys_ref[...] = lax.dot_general(h, w2_ref[0], (((1,), (1,)), ((), ())))


_mlp = pl.pallas_call(
    _mlp_body,
    grid_spec=pltpu.PrefetchScalarGridSpec(
        num_scalar_prefetch=2,
        grid=(G,),
        in_specs=[
            pl.BlockSpec((M, D), lambda g, be, ba: (g, 0)),
            pl.BlockSpec((1, HC, D), lambda g, be, ba: (be[g], 0, 0)),
            pl.BlockSpec((1, D, HC), lambda g, be, ba: (be[g], 0, 0)),
        ],
        out_specs=pl.BlockSpec((M, D), lambda g, be, ba: (g, 0)),
    ),
    out_shape=jax.ShapeDtypeStruct((GM, D), jnp.float32),
    compiler_params=pltpu.CompilerParams(
        dimension_semantics=("arbitrary",),
    ),
)


@functools.partial(
    pl.kernel, mesh=_mesh,
    compiler_params=_sc_params,
    out_type=jax.ShapeDtypeStruct((S, D), jnp.float32),
    scratch_types=[pltpu.VMEM((TPW // 2,), jnp.int32),
                   pltpu.VMEM((TPW // 2,), jnp.int32),
                   pltpu.VMEM((TPW // 2,), jnp.float32),
                   pltpu.VMEM((TPW // 2,), jnp.float32),
                   pltpu.VMEM((TPW // 2, D), jnp.float32),
                   pltpu.VMEM((TPW // 2, D), jnp.float32),
                   pltpu.VMEM((TPW // 2, D), jnp.float32),
                   pltpu.SemaphoreType.DMA])
def _combine(inv_hbm, w0_hbm, w1_hbm, ys_hbm, out_hbm,
             s0, s1, w0_v, w1_v, y0, y1, o_v, sem):
    wid = _wid()
    half = TPW // 2
    for c in range(2):
        t0 = wid * TPW + c * half
        pltpu.sync_copy(inv_hbm.at[pl.ds(t0, half)], s0)
        pltpu.sync_copy(inv_hbm.at[pl.ds(S + t0, half)], s1)
        c0 = pltpu.async_copy(ys_hbm.at[s0], y0, sem)
        c1 = pltpu.async_copy(ys_hbm.at[s1], y1, sem)
        pltpu.sync_copy(w0_hbm.at[pl.ds(t0, half)], w0_v)
        pltpu.sync_copy(w1_hbm.at[pl.ds(t0, half)], w1_v)
        c0.wait()
        c1.wait()

        def body(i, _):
            a = plsc.load_gather(w0_v, [jnp.full((16,), i, jnp.int32)])
            b = plsc.load_gather(w1_v, [jnp.full((16,), i, jnp.int32)])
            for jj in range(D // 16):
                sl = pl.ds(jj * 16, 16)
                o_v[i, sl] = a * y0[i, sl] + b * y1[i, sl]
            return 0

        lax.fori_loop(0, half, body, 0)
        pltpu.sync_copy(o_v, out_hbm.at[pl.ds(t0, half)])


@jax.jit
def _moe(xf, Wr, W1, W2):
    eids, wts = _router(xf, Wr)
    ep = eids[:, :2].reshape(P)
    xs, inv, be, ba = _dispatch(ep, xf)
    ys = _mlp(be, ba, xs, W1, W2)
    return _combine(inv, wts[:, 0], wts[:, 1], ys)


def kernel(x, Wr, W1, W2):
    b, s, d = x.shape
    out = _moe(x.reshape(s, d), Wr, W1, W2)
    return out.reshape(b, s, d)
